# Initial kernel scaffold; baseline (speedup 1.0000x reference)
import jax
import jax.numpy as jnp
from jax.experimental import pallas as pl


def kernel(hidden_states, gen_token_mask, Wg_und, Wu_und, Wd_und, Wg_gen, Wu_gen, Wd_gen):
    raise NotImplementedError("write your pallas kernel here")


# speedup vs baseline: 1.0620x; 1.0620x over previous
import functools

import jax
import jax.numpy as jnp
from jax.experimental import pallas as pl
from jax.experimental.pallas import tpu as pltpu

_TM = 512
_FB = 512
_FPAD = 5632


def _moe_body(nf, m_ref, x_ref, wg_u, wu_u, wd_u, wg_g, wu_g, wd_g, out_ref, acc):
    j = pl.program_id(1)

    @pl.when(j == 0)
    def _():
        acc[...] = jnp.zeros_like(acc)

    x = x_ref[...]
    m = m_ref[...]

    def half(wg, wu):
        g = jnp.dot(x, wg[...], preferred_element_type=jnp.float32)
        u = jnp.dot(x, wu[...], preferred_element_type=jnp.float32)
        return jax.nn.silu(g) * u

    h_g = half(wg_g, wu_g) * m
    h_u = half(wg_u, wu_u) * (1.0 - m)
    acc[...] += (
        jnp.dot(h_g.astype(jnp.bfloat16), wd_g[...], preferred_element_type=jnp.float32)
        + jnp.dot(h_u.astype(jnp.bfloat16), wd_u[...], preferred_element_type=jnp.float32)
    )

    @pl.when(j == nf - 1)
    def _():
        out_ref[...] = acc[...]


def kernel(hidden_states, gen_token_mask, Wg_und, Wu_und, Wd_und, Wg_gen, Wu_gen, Wd_gen):
    T, D = hidden_states.shape
    F = Wg_und.shape[1]
    bf = jnp.bfloat16

    x = hidden_states.astype(bf)
    m = gen_token_mask.astype(jnp.float32)[:, None]

    pad_f = _FPAD - F

    def colpad(w):
        return jnp.pad(w.astype(bf), ((0, 0), (0, pad_f)))

    def rowpad(w):
        return jnp.pad(w.astype(bf), ((0, pad_f), (0, 0)))

    wg_u, wu_u = colpad(Wg_und), colpad(Wu_und)
    wg_g, wu_g = colpad(Wg_gen), colpad(Wu_gen)
    wd_u, wd_g = rowpad(Wd_und), rowpad(Wd_gen)

    nb = T // _TM
    nf = _FPAD // _FB

    grid = (nb, nf)
    in_specs = [
        pl.BlockSpec((_TM, 1), lambda i, j: (i, 0)),
        pl.BlockSpec((_TM, D), lambda i, j: (i, 0)),
        pl.BlockSpec((D, _FB), lambda i, j: (0, j)),
        pl.BlockSpec((D, _FB), lambda i, j: (0, j)),
        pl.BlockSpec((_FB, D), lambda i, j: (j, 0)),
        pl.BlockSpec((D, _FB), lambda i, j: (0, j)),
        pl.BlockSpec((D, _FB), lambda i, j: (0, j)),
        pl.BlockSpec((_FB, D), lambda i, j: (j, 0)),
    ]
    out = pl.pallas_call(
        functools.partial(_moe_body, nf),
        grid=grid,
        in_specs=in_specs,
        out_specs=pl.BlockSpec((_TM, D), lambda i, j: (i, 0)),
        out_shape=jax.ShapeDtypeStruct((T, D), jnp.float32),
        scratch_shapes=[pltpu.VMEM((_TM, D), jnp.float32)],
        compiler_params=pltpu.CompilerParams(
            dimension_semantics=("parallel", "arbitrary"),
        ),
    )(m, x, wg_u, wu_u, wd_u, wg_g, wu_g, wd_g)
    return out
